# baseline (speedup 1.0000x reference)
import functools

import jax
import jax.numpy as jnp
from jax import lax
from jax.experimental import pallas as pl
from jax.experimental.pallas import tpu as pltpu
from jax.experimental.pallas import tpu_sc as plsc

_CHUNK = 2048


def _fold_table(wte, W, b):
    def body(wte_ref, w_ref, b_ref, out_ref):
        out_ref[...] = lax.dot_general(
            wte_ref[...], w_ref[...],
            (((1,), (1,)), ((), ())),
            preferred_element_type=jnp.float32) + b_ref[...]

    return pl.pallas_call(
        body,
        out_shape=jax.ShapeDtypeStruct(wte.shape, jnp.float32),
    )(wte, W, b.reshape(1, -1))


def _sc_gather(idx, table):
    n = idx.shape[0]
    d = table.shape[1]
    info = plsc.get_sparse_core_info()
    nw = info.num_cores * info.num_subcores
    tpw = n // nw
    nchunks = tpw // _CHUNK
    n_pack = n * d // 128
    c_pack = _CHUNK * d // 128

    mesh = plsc.VectorSubcoreMesh(core_axis_name="c", subcore_axis_name="s")

    @functools.partial(
        pl.kernel,
        out_type=jax.ShapeDtypeStruct((n_pack, 128), jnp.float32),
        mesh=mesh,
        scratch_types=[
            pltpu.VMEM((_CHUNK,), jnp.int32),
            pltpu.VMEM(table.shape, jnp.float32),
            pltpu.VMEM((c_pack, 128), jnp.float32),
        ],
        compiler_params=pltpu.CompilerParams(
            use_tc_tiling_on_sc=False, needs_layout_passes=False),
    )
    def k(idx_hbm, table_hbm, out_hbm, idx_v, table_v, rows_v):
        wid = lax.axis_index("s") * info.num_cores + lax.axis_index("c")
        tok0 = wid * tpw
        pltpu.sync_copy(table_hbm, table_v)
        lane = lax.iota(jnp.int32, 16)
        rowoff = lax.shift_right_logical(lane, 3)
        coloff = (lane & 7) * d

        def chunk(c, carry):
            base = tok0 + c * _CHUNK
            pltpu.sync_copy(idx_hbm.at[pl.ds(base, _CHUNK)], idx_v)

            def group(t, carry2):
                iv = idx_v[pl.ds(t * 16, 16)]
                row = t * 2 + rowoff
                for dd in range(d):
                    col = coloff + dd
                    g = plsc.load_gather(
                        table_v, [iv, jnp.full((16,), dd, jnp.int32)])
                    plsc.store_scatter(rows_v, [row, col], g)
                return carry2

            lax.fori_loop(0, _CHUNK // 16, group, 0)
            pltpu.sync_copy(rows_v, out_hbm.at[pl.ds(base * d // 128, c_pack)])
            return carry

        lax.fori_loop(0, nchunks, chunk, 0)

    return k(idx, table)


def kernel(x, wte, W, b):
    bsz, seq = x.shape
    d = wte.shape[1]
    table2 = _fold_table(wte, W, b)
    out = _sc_gather(x.reshape(-1).astype(jnp.int32), table2)
    return out.reshape(bsz, seq, d)


# speedup vs baseline: 1.2842x; 1.2842x over previous
import functools

import jax
import jax.numpy as jnp
from jax import lax
from jax.experimental import pallas as pl
from jax.experimental.pallas import tpu as pltpu
from jax.experimental.pallas import tpu_sc as plsc

_ROWS = 8
_SPLITS = ((0, 128), (128, 72))


def _fold_table(wte, W, b):
    def body(wte_ref, w_ref, b_ref, out_ref):
        out_ref[...] = lax.dot_general(
            wte_ref[...], w_ref[...],
            (((1,), (1,)), ((), ())),
            preferred_element_type=jnp.float32) + b_ref[...]

    return pl.pallas_call(
        body,
        out_shape=jax.ShapeDtypeStruct(wte.shape, jnp.float32),
    )(wte, W, b.reshape(1, -1))


def _sc_gather(idx, table):
    bsz, seq = idx.shape
    d = table.shape[1]
    info = plsc.get_sparse_core_info()
    nw = info.num_cores * info.num_subcores
    rpw = bsz // nw
    nchunks = rpw // _ROWS

    mesh = plsc.VectorSubcoreMesh(core_axis_name="c", subcore_axis_name="s")

    @functools.partial(
        pl.kernel,
        out_type=jax.ShapeDtypeStruct((bsz, seq, d), jnp.float32),
        mesh=mesh,
        scratch_types=[
            pltpu.VMEM((_ROWS, seq), jnp.int32),
            pltpu.VMEM((_ROWS, seq, d), jnp.float32),
            pltpu.SemaphoreType.DMA,
        ],
        compiler_params=pltpu.CompilerParams(use_tc_tiling_on_sc=False),
    )
    def k(idx_hbm, table_hbm, out_hbm, idx_v, rows_v, sem):
        wid = lax.axis_index("s") * info.num_cores + lax.axis_index("c")
        row0 = wid * rpw

        def chunk(c, carry):
            b0 = row0 + c * _ROWS
            pltpu.sync_copy(idx_hbm.at[pl.ds(b0, _ROWS)], idx_v)
            copies = [
                pltpu.async_copy(
                    table_hbm.at[idx_v.at[j, pl.ds(off, ln)]],
                    rows_v.at[j, pl.ds(off, ln)], sem)
                for j in range(_ROWS)
                for off, ln in _SPLITS
            ]
            for cp in copies:
                cp.wait()
            pltpu.sync_copy(rows_v, out_hbm.at[pl.ds(b0, _ROWS)])
            return carry

        lax.fori_loop(0, nchunks, chunk, 0)

    return k(idx, table)


def kernel(x, wte, W, b):
    table2 = _fold_table(wte, W, b)
    return _sc_gather(x.astype(jnp.int32), table2)
